# baseline (speedup 1.0000x reference)
import functools

import jax
import jax.numpy as jnp
from jax import lax
from jax.experimental import pallas as pl
from jax.experimental.pallas import tpu as pltpu
from jax.experimental.pallas import tpu_sc as plsc

_LANES = 16
_NBUF = 3


def _sc_cumsum_rows(x2d, seq, ch):
    rows, feat = x2d.shape
    info = plsc.get_sparse_core_info()
    nc, ns = info.num_cores, info.num_subcores
    nw = nc * ns
    fw = feat // nw
    assert feat % nw == 0 and fw % _LANES == 0
    nvec = fw // _LANES
    assert seq % ch == 0
    nchunks = seq // ch

    mesh = plsc.VectorSubcoreMesh(core_axis_name="c", subcore_axis_name="s")

    @functools.partial(
        pl.kernel,
        mesh=mesh,
        out_type=jax.ShapeDtypeStruct((seq, feat), jnp.float32),
        scratch_types=(
            [pltpu.VMEM((ch, fw), jnp.float32) for _ in range(_NBUF)]
            + [pltpu.SemaphoreType.DMA for _ in range(2 * _NBUF)]
        ),
        compiler_params=pltpu.CompilerParams(use_tc_tiling_on_sc=False),
    )
    def run(x_hbm, o_hbm, *scratch):
        bufs = scratch[:_NBUF]
        in_sems = scratch[_NBUF : 2 * _NBUF]
        out_sems = scratch[2 * _NBUF :]

        wid = lax.axis_index("s") * nc + lax.axis_index("c")
        f0 = wid * fw

        def src(ci):
            return x_hbm.at[pl.ds(ci * ch, ch), pl.ds(f0, fw)]

        def dst(ci):
            return o_hbm.at[pl.ds(ci * ch, ch), pl.ds(f0, fw)]

        def make_row_body(p):
            def row_body(t, accs):
                new = []
                for j in range(nvec):
                    a = accs[j] + bufs[p][t, pl.ds(j * _LANES, _LANES)]
                    bufs[p][t, pl.ds(j * _LANES, _LANES)] = a
                    new.append(a)
                return tuple(new)

            return row_body

        in_handles = [None] * nchunks
        out_handles = [None] * nchunks
        in_handles[0] = pltpu.async_copy(src(0), bufs[0], in_sems[0])
        accs = tuple(jnp.zeros((_LANES,), jnp.float32) for _ in range(nvec))
        for ci in range(nchunks):
            p = ci % _NBUF
            if ci + 1 < nchunks:
                q = (ci + 1) % _NBUF
                if ci - 2 >= 0:
                    out_handles[ci - 2].wait()
                in_handles[ci + 1] = pltpu.async_copy(src(ci + 1), bufs[q], in_sems[q])
            in_handles[ci].wait()
            accs = lax.fori_loop(0, ch, make_row_body(p), accs)
            out_handles[ci] = pltpu.async_copy(bufs[p], dst(ci), out_sems[p])
        out_handles[nchunks - 2].wait()
        out_handles[nchunks - 1].wait()

    return run(x2d)


def _tc_cumsum_batch(x, bsel, s_blk=512, f_blk=2048):
    b, s, f = x.shape
    sb, fb = s // s_blk, f // f_blk

    def body(x_ref, o_ref, carry_ref):
        si = pl.program_id(1)

        @pl.when(si == 0)
        def _():
            carry_ref[...] = jnp.zeros_like(carry_ref)

        blk = x_ref[0]
        r = lax.broadcasted_iota(jnp.int32, (s_blk, s_blk), 0)
        c = lax.broadcasted_iota(jnp.int32, (s_blk, s_blk), 1)
        tri = jnp.where(r >= c, 1.0, 0.0)
        out = (
            jax.lax.dot(tri, blk, precision=jax.lax.Precision.DEFAULT)
            + carry_ref[...]
        )
        o_ref[0] = out
        carry_ref[...] = out[s_blk - 1 :, :]

    return pl.pallas_call(
        body,
        grid=(fb, sb),
        in_specs=[
            pl.BlockSpec((1, s_blk, f_blk), lambda bf, si: (bsel, si, bf))
        ],
        out_specs=pl.BlockSpec((1, s_blk, f_blk), lambda bf, si: (0, si, bf)),
        out_shape=jax.ShapeDtypeStruct((1, s, f), jnp.float32),
        scratch_shapes=[pltpu.VMEM((1, f_blk), jnp.float32)],
        compiler_params=pltpu.CompilerParams(
            dimension_semantics=("arbitrary", "arbitrary")
        ),
    )(x)


def kernel(x, dim):
    del dim
    b, s, f = x.shape
    sc_out = _sc_cumsum_rows(x.reshape(b * s, f), seq=s, ch=512)
    tc_out = _tc_cumsum_batch(x, bsel=1)
    return jnp.concatenate([sc_out.reshape(1, s, f), tc_out], axis=0)


# speedup vs baseline: 1.8045x; 1.8045x over previous
import functools

import jax
import jax.numpy as jnp
from jax import lax
from jax.experimental import pallas as pl
from jax.experimental.pallas import tpu as pltpu
from jax.experimental.pallas import tpu_sc as plsc

_LANES = 16
_NBUF = 3


def _sc_cumsum_rows(x2d, seq, ch):
    rows, feat = x2d.shape
    info = plsc.get_sparse_core_info()
    nc, ns = info.num_cores, info.num_subcores
    fw = 128
    nstrips = feat // fw
    assert feat % fw == 0 and nstrips <= nc * ns
    nvec = fw // _LANES
    assert seq % ch == 0
    nchunks = seq // ch

    mesh = plsc.VectorSubcoreMesh(core_axis_name="c", subcore_axis_name="s")

    @functools.partial(
        pl.kernel,
        mesh=mesh,
        out_type=jax.ShapeDtypeStruct((seq, feat), jnp.float32),
        scratch_types=(
            [pltpu.VMEM((ch, fw), jnp.float32) for _ in range(_NBUF)]
            + [pltpu.SemaphoreType.DMA for _ in range(2 * _NBUF)]
        ),
    )
    def run(x_hbm, o_hbm, *scratch):
        bufs = scratch[:_NBUF]
        in_sems = scratch[_NBUF : 2 * _NBUF]
        out_sems = scratch[2 * _NBUF :]

        wid = lax.axis_index("s") * nc + lax.axis_index("c")
        f0 = pl.multiple_of(wid * fw, fw)

        @pl.when(wid < nstrips)
        def _active():
            def src(ci):
                return x_hbm.at[pl.ds(ci * ch, ch), pl.ds(f0, fw)]

            def dst(ci):
                return o_hbm.at[pl.ds(ci * ch, ch), pl.ds(f0, fw)]

            def make_row_body(p):
                def row_body(t, accs):
                    new = []
                    for j in range(nvec):
                        a = accs[j] + bufs[p][t, pl.ds(j * _LANES, _LANES)]
                        bufs[p][t, pl.ds(j * _LANES, _LANES)] = a
                        new.append(a)
                    return tuple(new)

                return row_body

            in_handles = [None] * nchunks
            out_handles = [None] * nchunks
            in_handles[0] = pltpu.async_copy(src(0), bufs[0], in_sems[0])
            accs = tuple(jnp.zeros((_LANES,), jnp.float32) for _ in range(nvec))
            for ci in range(nchunks):
                p = ci % _NBUF
                if ci + 1 < nchunks:
                    q = (ci + 1) % _NBUF
                    if ci - 2 >= 0:
                        out_handles[ci - 2].wait()
                    in_handles[ci + 1] = pltpu.async_copy(
                        src(ci + 1), bufs[q], in_sems[q]
                    )
                in_handles[ci].wait()
                accs = lax.fori_loop(0, ch, make_row_body(p), accs)
                out_handles[ci] = pltpu.async_copy(bufs[p], dst(ci), out_sems[p])
            out_handles[nchunks - 2].wait()
            out_handles[nchunks - 1].wait()

    return run(x2d)


def _tc_cumsum_batch(x, bsel, s_blk=512, f_blk=2048):
    b, s, f = x.shape
    sb, fb = s // s_blk, f // f_blk

    def body(x_ref, o_ref, carry_ref):
        si = pl.program_id(1)

        @pl.when(si == 0)
        def _():
            carry_ref[...] = jnp.zeros_like(carry_ref)

        blk = x_ref[0]
        r = lax.broadcasted_iota(jnp.int32, (s_blk, s_blk), 0)
        c = lax.broadcasted_iota(jnp.int32, (s_blk, s_blk), 1)
        tri = jnp.where(r >= c, 1.0, 0.0)
        out = (
            jax.lax.dot(tri, blk, precision=jax.lax.Precision.DEFAULT)
            + carry_ref[...]
        )
        o_ref[0] = out
        carry_ref[...] = out[s_blk - 1 :, :]

    return pl.pallas_call(
        body,
        grid=(fb, sb),
        in_specs=[
            pl.BlockSpec((1, s_blk, f_blk), lambda bf, si: (bsel, si, bf))
        ],
        out_specs=pl.BlockSpec((1, s_blk, f_blk), lambda bf, si: (0, si, bf)),
        out_shape=jax.ShapeDtypeStruct((1, s, f), jnp.float32),
        scratch_shapes=[pltpu.VMEM((1, f_blk), jnp.float32)],
        compiler_params=pltpu.CompilerParams(
            dimension_semantics=("arbitrary", "arbitrary")
        ),
    )(x)


def kernel(x, dim):
    del dim
    b, s, f = x.shape
    sc_out = _sc_cumsum_rows(x.reshape(b * s, f), seq=s, ch=256)
    tc_out = _tc_cumsum_batch(x, bsel=1)
    return jnp.concatenate([sc_out.reshape(1, s, f), tc_out], axis=0)


# speedup vs baseline: 2.9875x; 1.6556x over previous
import functools

import jax
import jax.numpy as jnp
from jax import lax
from jax.experimental import pallas as pl
from jax.experimental.pallas import tpu as pltpu
from jax.experimental.pallas import tpu_sc as plsc

_LANES = 16
_NBUF = 3
_UNROLL = 8


def _sc_cumsum_2d(x2d, batch, seq):
    rows, feat = x2d.shape
    info = plsc.get_sparse_core_info()
    nc, ns = info.num_cores, info.num_subcores
    nw = nc * ns
    strips_per_batch = nw // batch
    fw = feat // strips_per_batch
    assert feat % strips_per_batch == 0 and fw % _LANES == 0
    nvec = fw // _LANES
    ch = 256
    assert seq % ch == 0
    nchunks = seq // ch

    mesh = plsc.VectorSubcoreMesh(core_axis_name="c", subcore_axis_name="s")

    @functools.partial(
        pl.kernel,
        mesh=mesh,
        out_type=jax.ShapeDtypeStruct((rows, feat), jnp.float32),
        scratch_types=(
            [pltpu.VMEM((ch, fw), jnp.float32) for _ in range(_NBUF)]
            + [pltpu.SemaphoreType.DMA for _ in range(2 * _NBUF)]
        ),
    )
    def run(x_hbm, o_hbm, *scratch):
        bufs = scratch[:_NBUF]
        in_sems = scratch[_NBUF : 2 * _NBUF]
        out_sems = scratch[2 * _NBUF :]

        wid = lax.axis_index("s") * nc + lax.axis_index("c")
        b = wid // strips_per_batch
        f0 = pl.multiple_of((wid % strips_per_batch) * fw, fw)
        row0 = b * seq

        def src(ci):
            return x_hbm.at[pl.ds(row0 + ci * ch, ch), pl.ds(f0, fw)]

        def dst(ci):
            return o_hbm.at[pl.ds(row0 + ci * ch, ch), pl.ds(f0, fw)]

        def make_row_body(p):
            def row_body(t, accs):
                new = accs
                for u in range(_UNROLL):
                    cur = []
                    for j in range(nvec):
                        a = new[j] + bufs[p][t * _UNROLL + u, pl.ds(j * _LANES, _LANES)]
                        bufs[p][t * _UNROLL + u, pl.ds(j * _LANES, _LANES)] = a
                        cur.append(a)
                    new = tuple(cur)
                return new

            return row_body

        in_handles = [None] * nchunks
        out_handles = [None] * nchunks
        in_handles[0] = pltpu.async_copy(src(0), bufs[0], in_sems[0])
        accs = tuple(jnp.zeros((_LANES,), jnp.float32) for _ in range(nvec))
        for ci in range(nchunks):
            p = ci % _NBUF
            if ci + 1 < nchunks:
                q = (ci + 1) % _NBUF
                if ci - 2 >= 0:
                    out_handles[ci - 2].wait()
                in_handles[ci + 1] = pltpu.async_copy(src(ci + 1), bufs[q], in_sems[q])
            in_handles[ci].wait()
            accs = lax.fori_loop(0, ch // _UNROLL, make_row_body(p), accs)
            out_handles[ci] = pltpu.async_copy(bufs[p], dst(ci), out_sems[p])
        out_handles[nchunks - 2].wait()
        out_handles[nchunks - 1].wait()

    return run(x2d)


def kernel(x, dim):
    del dim
    b, s, f = x.shape
    out = _sc_cumsum_2d(x.reshape(b * s, f), b, s)
    return out.reshape(b, s, f)
